# Initial kernel scaffold; baseline (speedup 1.0000x reference)
import jax
import jax.numpy as jnp
from jax.experimental import pallas as pl


def kernel(prob):
    raise NotImplementedError("write your pallas kernel here")


# speedup vs baseline: 12.6041x; 12.6041x over previous
import functools

import jax
import jax.numpy as jnp
from jax.experimental import pallas as pl
from jax.experimental.pallas import tpu as pltpu

_N, _C, _H, _W = 4, 19, 512, 1024
_BH = 8
_RATIO = 0.2


def _pass_kernel(x_ref, out_ref, cnt_acc, sum_acc):
    n = pl.program_id(0)
    h = pl.program_id(1)

    @pl.when(jnp.logical_and(n == 0, h == 0))
    def _init():
        cnt_acc[...] = jnp.zeros_like(cnt_acc)
        sum_acc[...] = jnp.zeros_like(sum_acc)

    x = x_ref[0]
    cur = x[0]
    idx = jnp.zeros(cur.shape, jnp.int32)
    s = cur * cur
    for c in range(1, _C):
        xc = x[c]
        gt = xc > cur
        cur = jnp.where(gt, xc, cur)
        idx = jnp.where(gt, c, idx)
        s = s + xc * xc

    cls = jax.lax.broadcasted_iota(jnp.int32, (_C, _BH, _W), 0)
    onehot = idx[None] == cls
    cnt = jnp.sum(onehot.astype(jnp.float32), axis=(1, 2))
    ssum = jnp.sum(jnp.where(onehot, s[None], 0.0), axis=(1, 2))

    row = jax.lax.broadcasted_iota(jnp.int32, (8, _C), 0)
    cnt_acc[...] += jnp.where(row == n, cnt[None, :], 0.0)
    sum_acc[...] += jnp.where(row == n, ssum[None, :], 0.0)

    @pl.when(jnp.logical_and(n == _N - 1, h == pl.num_programs(1) - 1))
    def _finish():
        hc = cnt_acc[...]
        hs = sum_acc[...]
        total = jnp.sum(hc, axis=1, keepdims=True)
        denom = jnp.maximum(
            jnp.power(hc, _RATIO) * jnp.power(total, 1.0 - _RATIO), 1.0
        )
        out_ref[0, 0] = -jnp.sum(hs / denom) / (_N * _C)


def kernel(prob):
    out = pl.pallas_call(
        _pass_kernel,
        grid=(_N, _H // _BH),
        in_specs=[
            pl.BlockSpec((1, _C, _BH, _W), lambda n, h: (n, 0, h, 0)),
        ],
        out_specs=pl.BlockSpec(memory_space=pltpu.SMEM),
        out_shape=jax.ShapeDtypeStruct((1, 1), jnp.float32),
        scratch_shapes=[
            pltpu.VMEM((8, _C), jnp.float32),
            pltpu.VMEM((8, _C), jnp.float32),
        ],
    )(prob)
    return out[0, 0]
